# baseline (speedup 1.0000x reference)
import jax
import jax.numpy as jnp
from jax.experimental import pallas as pl
from jax.experimental.pallas import tpu as pltpu

_N = 22500
_T = 22
_NPAD = _T * 8 * 128
_MAX_OUT = 300
_IOU_THR = 0.7
_IMG = 800.0


def _pick(a, b):
    take_b = (b[0] > a[0]) | ((b[0] == a[0]) & (b[1] < a[1]))
    return tuple(jnp.where(take_b, y, x) for x, y in zip(a, b))


def _nms_body(scores_ref, reg_ref, anc_ref, out_ref, box_ref, sel_ref):
    f0 = jnp.float32(0.0)
    x1a = anc_ref[0]
    y1a = anc_ref[1]
    x2a = anc_ref[2]
    y2a = anc_ref[3]
    wa = x2a - x1a
    ha = y2a - y1a
    cxa = x1a + wa * 0.5
    cya = y1a + ha * 0.5
    cx = reg_ref[0] * wa + cxa
    cy = reg_ref[1] * ha + cya
    w = wa * jnp.exp(reg_ref[2])
    h = ha * jnp.exp(reg_ref[3])
    x1 = jnp.minimum(jnp.maximum(cx - w * 0.5, f0), _IMG)
    y1 = jnp.minimum(jnp.maximum(cy - h * 0.5, f0), _IMG)
    x2 = jnp.minimum(jnp.maximum(cx + w * 0.5, f0), _IMG)
    y2 = jnp.minimum(jnp.maximum(cy + h * 0.5, f0), _IMG)
    box_ref[0] = x1
    box_ref[1] = y1
    box_ref[2] = x2
    box_ref[3] = y2
    box_ref[4] = (x2 - x1) * (y2 - y1)

    it = jax.lax.broadcasted_iota
    rowi = (it(jnp.int32, (_T, 8, 128), 0) * 8
            + it(jnp.int32, (_T, 8, 128), 1))
    lane = it(jnp.int32, (1, 128), 1)
    lanef = lane.astype(jnp.float32)

    def step(i, s):
        x1c = box_ref[0]
        y1c = box_ref[1]
        x2c = box_ref[2]
        y2c = box_ref[3]
        items = [(s[t], rowi[t], x1c[t], y1c[t], x2c[t], y2c[t])
                 for t in range(_T)]
        while len(items) > 1:
            nxt = [_pick(items[j], items[j + 1])
                   for j in range(0, len(items) - 1, 2)]
            if len(items) % 2:
                nxt.append(items[-1])
            items = nxt
        cur = items[0]
        for sh in (1, 2, 4):
            cur = _pick(cur, tuple(pltpu.roll(v, sh, 0) for v in cur))
        vrow = cur[0][0:1]
        m = jnp.max(vrow, keepdims=True)
        mask = vrow == m
        cnt = jnp.sum(jnp.where(mask, 1.0, f0), keepdims=True)
        x1s = jnp.sum(jnp.where(mask, cur[2][0:1], f0), keepdims=True)
        y1s = jnp.sum(jnp.where(mask, cur[3][0:1], f0), keepdims=True)
        x2s = jnp.sum(jnp.where(mask, cur[4][0:1], f0), keepdims=True)
        y2s = jnp.sum(jnp.where(mask, cur[5][0:1], f0), keepdims=True)
        sel_ref[0:1, :] = jnp.broadcast_to(x1s, (1, 128))
        sel_ref[1:2, :] = jnp.broadcast_to(y1s, (1, 128))
        sel_ref[2:3, :] = jnp.broadcast_to(x2s, (1, 128))
        sel_ref[3:4, :] = jnp.broadcast_to(y2s, (1, 128))
        sel_ref[4:5, :] = jnp.broadcast_to(m, (1, 128))

        @pl.when(cnt[0, 0] > 1.0)
        def _tie_fix():
            lw = jnp.min(jnp.where(mask, lanef, 1e9), keepdims=True)
            onehot = lanef == lw
            sel_ref[0:1, :] = jnp.broadcast_to(jnp.sum(
                jnp.where(onehot, cur[2][0:1], f0), keepdims=True), (1, 128))
            sel_ref[1:2, :] = jnp.broadcast_to(jnp.sum(
                jnp.where(onehot, cur[3][0:1], f0), keepdims=True), (1, 128))
            sel_ref[2:3, :] = jnp.broadcast_to(jnp.sum(
                jnp.where(onehot, cur[4][0:1], f0), keepdims=True), (1, 128))
            sel_ref[3:4, :] = jnp.broadcast_to(jnp.sum(
                jnp.where(onehot, cur[5][0:1], f0), keepdims=True), (1, 128))
            ss_slow = jnp.where(
                m == jnp.float32(-1e9),
                jnp.sum(jnp.where(lane == 0, scores_ref[0, 0:1, :], f0),
                        keepdims=True),
                m)
            sel_ref[4:5, :] = jnp.broadcast_to(ss_slow, (1, 128))

        x1b = sel_ref[0:1, :]
        y1b = sel_ref[1:2, :]
        x2b = sel_ref[2:3, :]
        y2b = sel_ref[3:4, :]
        ssb = sel_ref[4:5, :]
        area_s = (x2b - x1b) * (y2b - y1b)
        xx1 = jnp.maximum(x1c, x1b[None])
        yy1 = jnp.maximum(y1c, y1b[None])
        xx2 = jnp.minimum(x2c, x2b[None])
        yy2 = jnp.minimum(y2c, y2b[None])
        inter = jnp.maximum(xx2 - xx1, f0) * jnp.maximum(yy2 - yy1, f0)
        iou = inter / (box_ref[4] + area_s[None] - inter + 1e-9)
        s2 = jnp.where(iou >= _IOU_THR, -1e9, s)
        row = jnp.where(lane == 0, x1b,
              jnp.where(lane == 1, y1b,
              jnp.where(lane == 2, x2b,
              jnp.where(lane == 3, y2b, ssb))))
        out_ref[pl.ds(i, 1), :] = row
        return s2

    jax.lax.fori_loop(0, _MAX_OUT, step, scores_ref[...])


def _to_colmajor(a):
    return a.reshape(128, _T, 8).transpose(1, 2, 0)


def kernel(cls_output, reg_output, anchors):
    f32 = jnp.float32
    pad = _NPAD - _N
    scores = _to_colmajor(jnp.concatenate(
        [cls_output.astype(f32), jnp.full((pad,), -jnp.inf, f32)]))
    reg_p = jnp.concatenate(
        [reg_output.astype(f32), jnp.zeros((pad, 4), f32)]).T
    anc_p = jnp.concatenate(
        [anchors.astype(f32), jnp.zeros((pad, 4), f32)]).T
    reg4 = jnp.stack([_to_colmajor(reg_p[k]) for k in range(4)])
    anc4 = jnp.stack([_to_colmajor(anc_p[k]) for k in range(4)])

    out = pl.pallas_call(
        _nms_body,
        out_shape=jax.ShapeDtypeStruct((304, 128), f32),
        scratch_shapes=[
            pltpu.VMEM((5, _T, 8, 128), f32),
            pltpu.VMEM((8, 128), f32),
        ],
    )(scores, reg4, anc4)

    rois = out[:_MAX_OUT, 0:4]
    roi_scores = out[:_MAX_OUT, 4]
    return roi_scores, rois


# speedup vs baseline: 1.0000x; 1.0000x over previous
import jax
import jax.numpy as jnp
from jax.experimental import pallas as pl
from jax.experimental.pallas import tpu as pltpu

_N = 22500
_T = 22
_NPAD = _T * 8 * 128
_MAX_OUT = 300
_IOU_THR = 0.7
_IMG = 800.0


def _pick(a, b):
    take_b = (b[0] > a[0]) | ((b[0] == a[0]) & (b[1] < a[1]))
    return tuple(jnp.where(take_b, y, x) for x, y in zip(a, b))


def _nms_body(scores_ref, reg_ref, anc_ref, out_ref, box_ref, sel_ref):
    f0 = jnp.float32(0.0)
    x1a = anc_ref[0]
    y1a = anc_ref[1]
    x2a = anc_ref[2]
    y2a = anc_ref[3]
    wa = x2a - x1a
    ha = y2a - y1a
    cxa = x1a + wa * 0.5
    cya = y1a + ha * 0.5
    cx = reg_ref[0] * wa + cxa
    cy = reg_ref[1] * ha + cya
    w = wa * jnp.exp(reg_ref[2])
    h = ha * jnp.exp(reg_ref[3])
    x1 = jnp.minimum(jnp.maximum(cx - w * 0.5, f0), _IMG)
    y1 = jnp.minimum(jnp.maximum(cy - h * 0.5, f0), _IMG)
    x2 = jnp.minimum(jnp.maximum(cx + w * 0.5, f0), _IMG)
    y2 = jnp.minimum(jnp.maximum(cy + h * 0.5, f0), _IMG)
    box_ref[0] = x1
    box_ref[1] = y1
    box_ref[2] = x2
    box_ref[3] = y2
    box_ref[4] = (x2 - x1) * (y2 - y1)

    it = jax.lax.broadcasted_iota
    rowi = (it(jnp.int32, (_T, 8, 128), 0) * 8
            + it(jnp.int32, (_T, 8, 128), 1))
    lane = it(jnp.int32, (1, 128), 1)
    lanef = lane.astype(jnp.float32)

    def step(i, s):
        x1c = box_ref[0]
        y1c = box_ref[1]
        x2c = box_ref[2]
        y2c = box_ref[3]
        items = [(s[t], rowi[t], x1c[t], y1c[t], x2c[t], y2c[t])
                 for t in range(_T)]
        while len(items) > 1:
            nxt = [_pick(items[j], items[j + 1])
                   for j in range(0, len(items) - 1, 2)]
            if len(items) % 2:
                nxt.append(items[-1])
            items = nxt
        cur = items[0]
        for sh in (1, 2, 4):
            cur = _pick(cur, tuple(pltpu.roll(v, sh, 0) for v in cur))
        vrow = cur[0][0:1]
        m = jnp.max(vrow, keepdims=True)
        mask = vrow >= m
        cnt = jnp.sum(jnp.where(mask, 1.0, f0), keepdims=True)
        x1s = jnp.sum(jnp.where(mask, cur[2][0:1], f0), keepdims=True)
        y1s = jnp.sum(jnp.where(mask, cur[3][0:1], f0), keepdims=True)
        x2s = jnp.sum(jnp.where(mask, cur[4][0:1], f0), keepdims=True)
        y2s = jnp.sum(jnp.where(mask, cur[5][0:1], f0), keepdims=True)
        sel_ref[0:1, :] = jnp.broadcast_to(x1s, (1, 128))
        sel_ref[1:2, :] = jnp.broadcast_to(y1s, (1, 128))
        sel_ref[2:3, :] = jnp.broadcast_to(x2s, (1, 128))
        sel_ref[3:4, :] = jnp.broadcast_to(y2s, (1, 128))
        sel_ref[4:5, :] = jnp.broadcast_to(m, (1, 128))

        @pl.when(cnt[0, 0] > 1.0)
        def _tie_fix():
            lw = jnp.min(jnp.where(mask, lanef, 1e9), keepdims=True)
            onehot = lanef == lw
            sel_ref[0:1, :] = jnp.broadcast_to(jnp.sum(
                jnp.where(onehot, cur[2][0:1], f0), keepdims=True), (1, 128))
            sel_ref[1:2, :] = jnp.broadcast_to(jnp.sum(
                jnp.where(onehot, cur[3][0:1], f0), keepdims=True), (1, 128))
            sel_ref[2:3, :] = jnp.broadcast_to(jnp.sum(
                jnp.where(onehot, cur[4][0:1], f0), keepdims=True), (1, 128))
            sel_ref[3:4, :] = jnp.broadcast_to(jnp.sum(
                jnp.where(onehot, cur[5][0:1], f0), keepdims=True), (1, 128))
            ss_slow = jnp.where(
                m == jnp.float32(-1e9),
                jnp.sum(jnp.where(lane == 0, scores_ref[0, 0:1, :], f0),
                        keepdims=True),
                m)
            sel_ref[4:5, :] = jnp.broadcast_to(ss_slow, (1, 128))

        x1b = sel_ref[0:1, :]
        y1b = sel_ref[1:2, :]
        x2b = sel_ref[2:3, :]
        y2b = sel_ref[3:4, :]
        ssb = sel_ref[4:5, :]
        area_s = (x2b - x1b) * (y2b - y1b)
        xx1 = jnp.maximum(x1c, x1b[None])
        yy1 = jnp.maximum(y1c, y1b[None])
        xx2 = jnp.minimum(x2c, x2b[None])
        yy2 = jnp.minimum(y2c, y2b[None])
        inter = jnp.maximum(xx2 - xx1, f0) * jnp.maximum(yy2 - yy1, f0)
        iou = inter / (box_ref[4] + area_s[None] - inter + 1e-9)
        s2 = jnp.where(iou >= _IOU_THR, -1e9, s)
        row = jnp.where(lane == 0, x1b,
              jnp.where(lane == 1, y1b,
              jnp.where(lane == 2, x2b,
              jnp.where(lane == 3, y2b, ssb))))
        out_ref[pl.ds(i, 1), :] = row
        return s2

    jax.lax.fori_loop(0, _MAX_OUT, step, scores_ref[...])


def _to_colmajor(a):
    return a.reshape(128, _T, 8).transpose(1, 2, 0)


def kernel(cls_output, reg_output, anchors):
    f32 = jnp.float32
    pad = _NPAD - _N
    scores = _to_colmajor(jnp.concatenate(
        [cls_output.astype(f32), jnp.full((pad,), -jnp.inf, f32)]))
    reg_p = jnp.concatenate(
        [reg_output.astype(f32), jnp.zeros((pad, 4), f32)]).T
    anc_p = jnp.concatenate(
        [anchors.astype(f32), jnp.zeros((pad, 4), f32)]).T
    reg4 = jnp.stack([_to_colmajor(reg_p[k]) for k in range(4)])
    anc4 = jnp.stack([_to_colmajor(anc_p[k]) for k in range(4)])

    out = pl.pallas_call(
        _nms_body,
        out_shape=jax.ShapeDtypeStruct((304, 128), f32),
        scratch_shapes=[
            pltpu.VMEM((5, _T, 8, 128), f32),
            pltpu.VMEM((8, 128), f32),
        ],
    )(scores, reg4, anc4)

    rois = out[:_MAX_OUT, 0:4]
    roi_scores = out[:_MAX_OUT, 4]
    return roi_scores, rois


# speedup vs baseline: 1.2316x; 1.2315x over previous
import jax
import jax.numpy as jnp
from jax.experimental import pallas as pl
from jax.experimental.pallas import tpu as pltpu

_N = 22500
_T = 22
_NPAD = _T * 8 * 128
_MAX_OUT = 300
_IOU_THR = 0.7
_IMG = 800.0


def _merge(a, b):
    take_b = (b[0] > a[0]) | ((b[0] == a[0]) & (b[1] < a[1]))
    v2 = jnp.maximum(jnp.minimum(a[0], b[0]),
                     jnp.where(a[0] >= b[0], a[6], b[6]))
    out = tuple(jnp.where(take_b, y, x) for x, y in zip(a[:6], b[:6]))
    return out + (v2,)


def _nms_body(scores_ref, reg_ref, anc_ref, out_ref, box_ref, sel_ref):
    f0 = jnp.float32(0.0)
    neg = jnp.float32(-1e9)
    ninf = jnp.float32(-jnp.inf)
    x1a = anc_ref[0]
    y1a = anc_ref[1]
    x2a = anc_ref[2]
    y2a = anc_ref[3]
    wa = x2a - x1a
    ha = y2a - y1a
    cxa = x1a + wa * 0.5
    cya = y1a + ha * 0.5
    cx = reg_ref[0] * wa + cxa
    cy = reg_ref[1] * ha + cya
    w = wa * jnp.exp(reg_ref[2])
    h = ha * jnp.exp(reg_ref[3])
    x1 = jnp.minimum(jnp.maximum(cx - w * 0.5, f0), _IMG)
    y1 = jnp.minimum(jnp.maximum(cy - h * 0.5, f0), _IMG)
    x2 = jnp.minimum(jnp.maximum(cx + w * 0.5, f0), _IMG)
    y2 = jnp.minimum(jnp.maximum(cy + h * 0.5, f0), _IMG)
    box_ref[0] = x1
    box_ref[1] = y1
    box_ref[2] = x2
    box_ref[3] = y2
    box_ref[4] = (x2 - x1) * (y2 - y1)

    it = jax.lax.broadcasted_iota
    rowi = (it(jnp.int32, (_T, 8, 128), 0) * 8
            + it(jnp.int32, (_T, 8, 128), 1))
    lane = it(jnp.int32, (1, 128), 1)
    lanef = lane.astype(jnp.float32)
    zleaf = jnp.full((8, 128), -jnp.inf, jnp.float32)

    def body(carry):
        n, s = carry
        x1c = box_ref[0]
        y1c = box_ref[1]
        x2c = box_ref[2]
        y2c = box_ref[3]
        items = [(s[t], rowi[t], x1c[t], y1c[t], x2c[t], y2c[t], zleaf)
                 for t in range(_T)]
        while len(items) > 1:
            nxt = [_merge(items[j], items[j + 1])
                   for j in range(0, len(items) - 1, 2)]
            if len(items) % 2:
                nxt.append(items[-1])
            items = nxt
        cur = items[0]
        for sh in (1, 2, 4):
            cur = _merge(cur, tuple(pltpu.roll(v, sh, 0) for v in cur))
        vrow = cur[0][0:1]
        v2row = cur[6][0:1]
        tx1 = cur[2][0:1]
        ty1 = cur[3][0:1]
        tx2 = cur[4][0:1]
        ty2 = cur[5][0:1]
        m = jnp.max(vrow, keepdims=True)
        mask = vrow >= m
        cnt = jnp.sum(jnp.where(mask, 1.0, f0), keepdims=True)
        x1s = jnp.sum(jnp.where(mask, tx1, f0), keepdims=True)
        y1s = jnp.sum(jnp.where(mask, ty1, f0), keepdims=True)
        x2s = jnp.sum(jnp.where(mask, tx2, f0), keepdims=True)
        y2s = jnp.sum(jnp.where(mask, ty2, f0), keepdims=True)
        m2 = jnp.max(jnp.where(mask, ninf, vrow), keepdims=True)
        hid1 = jnp.sum(jnp.where(mask, v2row, f0), keepdims=True)
        sel_ref[0:1, :] = jnp.broadcast_to(x1s, (1, 128))
        sel_ref[1:2, :] = jnp.broadcast_to(y1s, (1, 128))
        sel_ref[2:3, :] = jnp.broadcast_to(x2s, (1, 128))
        sel_ref[3:4, :] = jnp.broadcast_to(y2s, (1, 128))
        sel_ref[4:5, :] = jnp.broadcast_to(m, (1, 128))

        @pl.when(cnt[0, 0] > 1.0)
        def _tie_fix():
            lw = jnp.min(jnp.where(mask, lanef, 1e9), keepdims=True)
            onehot = lanef == lw
            sel_ref[0:1, :] = jnp.broadcast_to(jnp.sum(
                jnp.where(onehot, tx1, f0), keepdims=True), (1, 128))
            sel_ref[1:2, :] = jnp.broadcast_to(jnp.sum(
                jnp.where(onehot, ty1, f0), keepdims=True), (1, 128))
            sel_ref[2:3, :] = jnp.broadcast_to(jnp.sum(
                jnp.where(onehot, tx2, f0), keepdims=True), (1, 128))
            sel_ref[3:4, :] = jnp.broadcast_to(jnp.sum(
                jnp.where(onehot, ty2, f0), keepdims=True), (1, 128))
            ss_slow = jnp.where(
                m == neg,
                jnp.sum(jnp.where(lane == 0, scores_ref[0, 0:1, :], f0),
                        keepdims=True),
                m)
            sel_ref[4:5, :] = jnp.broadcast_to(ss_slow, (1, 128))

        x1b = sel_ref[0:1, :]
        y1b = sel_ref[1:2, :]
        x2b = sel_ref[2:3, :]
        y2b = sel_ref[3:4, :]
        ssb = sel_ref[4:5, :]
        area_s = (x2b - x1b) * (y2b - y1b)
        art = (tx2 - tx1) * (ty2 - ty1)
        xxr = jnp.maximum(tx1, x1b)
        yyr = jnp.maximum(ty1, y1b)
        xx2r = jnp.minimum(tx2, x2b)
        yy2r = jnp.minimum(ty2, y2b)
        interr = jnp.maximum(xx2r - xxr, f0) * jnp.maximum(yy2r - yyr, f0)
        iour = interr / (art + area_s - interr + 1e-9)
        suppr = iour >= _IOU_THR
        self_dead = jnp.sum(jnp.where(mask & suppr, 1.0, f0), keepdims=True)
        hcap = jnp.max(jnp.where(suppr & (~mask), vrow, ninf), keepdims=True)
        mask2 = (~mask) & (vrow >= m2)
        cnt2 = jnp.sum(jnp.where(mask2, 1.0, f0), keepdims=True)
        x1q = jnp.sum(jnp.where(mask2, tx1, f0), keepdims=True)
        y1q = jnp.sum(jnp.where(mask2, ty1, f0), keepdims=True)
        x2q = jnp.sum(jnp.where(mask2, tx2, f0), keepdims=True)
        y2q = jnp.sum(jnp.where(mask2, ty2, f0), keepdims=True)
        ar2 = (x2q - x1q) * (y2q - y1q)
        xx12 = jnp.maximum(x1q, x1b[0:1, 0:1])
        yy12 = jnp.maximum(y1q, y1b[0:1, 0:1])
        xx22 = jnp.minimum(x2q, x2b[0:1, 0:1])
        yy22 = jnp.minimum(y2q, y2b[0:1, 0:1])
        inter12 = (jnp.maximum(xx22 - xx12, f0)
                   * jnp.maximum(yy22 - yy12, f0))
        area_s11 = area_s[0:1, 0:1]
        iou12 = inter12 / (ar2 + area_s11 - inter12 + 1e-9)
        valid2 = ((cnt == 1.0) & (cnt2 == 1.0) & (m2 > neg)
                  & (m2 > hcap) & (m2 > hid1) & (iou12 < _IOU_THR)
                  & (self_dead == 1.0))
        xx1 = jnp.maximum(x1c, x1b[None])
        yy1 = jnp.maximum(y1c, y1b[None])
        xx2 = jnp.minimum(x2c, x2b[None])
        yy2 = jnp.minimum(y2c, y2b[None])
        inter = jnp.maximum(xx2 - xx1, f0) * jnp.maximum(yy2 - yy1, f0)
        iou = inter / (box_ref[4] + area_s[None] - inter + 1e-9)
        xx1q = jnp.maximum(x1c, x1q[None])
        yy1q = jnp.maximum(y1c, y1q[None])
        xx2q = jnp.minimum(x2c, x2q[None])
        yy2q = jnp.minimum(y2c, y2q[None])
        interq = jnp.maximum(xx2q - xx1q, f0) * jnp.maximum(yy2q - yy1q, f0)
        iouq = interq / (box_ref[4] + ar2[None] - interq + 1e-9)
        kill = (iou >= _IOU_THR) | (valid2[None] & (iouq >= _IOU_THR))
        s2 = jnp.where(kill, neg, s)
        row1 = jnp.where(lane == 0, x1b,
               jnp.where(lane == 1, y1b,
               jnp.where(lane == 2, x2b,
               jnp.where(lane == 3, y2b, ssb))))
        row2 = jnp.where(lane == 0, x1q,
               jnp.where(lane == 1, y1q,
               jnp.where(lane == 2, x2q,
               jnp.where(lane == 3, y2q, m2))))
        out_ref[pl.ds(n, 1), :] = row1
        out_ref[pl.ds(n + 1, 1), :] = row2
        n2 = n + 1 + jnp.where(valid2, 1, 0)[0, 0]
        return (n2, s2)

    jax.lax.while_loop(lambda c: c[0] < _MAX_OUT, body,
                       (jnp.int32(0), scores_ref[...]))


def _to_colmajor(a):
    return a.reshape(128, _T, 8).transpose(1, 2, 0)


def kernel(cls_output, reg_output, anchors):
    f32 = jnp.float32
    pad = _NPAD - _N
    scores = _to_colmajor(jnp.concatenate(
        [cls_output.astype(f32), jnp.full((pad,), -jnp.inf, f32)]))
    reg_p = jnp.concatenate(
        [reg_output.astype(f32), jnp.zeros((pad, 4), f32)]).T
    anc_p = jnp.concatenate(
        [anchors.astype(f32), jnp.zeros((pad, 4), f32)]).T
    reg4 = jnp.stack([_to_colmajor(reg_p[k]) for k in range(4)])
    anc4 = jnp.stack([_to_colmajor(anc_p[k]) for k in range(4)])

    out = pl.pallas_call(
        _nms_body,
        out_shape=jax.ShapeDtypeStruct((304, 128), f32),
        scratch_shapes=[
            pltpu.VMEM((5, _T, 8, 128), f32),
            pltpu.VMEM((8, 128), f32),
        ],
    )(scores, reg4, anc4)

    rois = out[:_MAX_OUT, 0:4]
    roi_scores = out[:_MAX_OUT, 4]
    return roi_scores, rois


# speedup vs baseline: 1.3670x; 1.1100x over previous
import jax
import jax.numpy as jnp
from jax.experimental import pallas as pl
from jax.experimental.pallas import tpu as pltpu

_N = 22500
_T = 22
_NPAD = _T * 8 * 128
_MAX_OUT = 300
_IOU_THR = 0.7
_IMG = 800.0


def _merge(a, b):
    take_b = (b[0] > a[0]) | ((b[0] == a[0]) & (b[1] < a[1]))
    v2 = jnp.maximum(jnp.minimum(a[0], b[0]),
                     jnp.where(a[0] >= b[0], a[6], b[6]))
    out = tuple(jnp.where(take_b, y, x) for x, y in zip(a[:6], b[:6]))
    return out + (v2,)


def _nms_body(scores_ref, reg_ref, anc_ref, out_ref, box_ref, sel_ref):
    f0 = jnp.float32(0.0)
    neg = jnp.float32(-1e9)
    ninf = jnp.float32(-jnp.inf)
    x1a = anc_ref[0]
    y1a = anc_ref[1]
    x2a = anc_ref[2]
    y2a = anc_ref[3]
    wa = x2a - x1a
    ha = y2a - y1a
    cxa = x1a + wa * 0.5
    cya = y1a + ha * 0.5
    cx = reg_ref[0] * wa + cxa
    cy = reg_ref[1] * ha + cya
    w = wa * jnp.exp(reg_ref[2])
    h = ha * jnp.exp(reg_ref[3])
    x1 = jnp.minimum(jnp.maximum(cx - w * 0.5, f0), _IMG)
    y1 = jnp.minimum(jnp.maximum(cy - h * 0.5, f0), _IMG)
    x2 = jnp.minimum(jnp.maximum(cx + w * 0.5, f0), _IMG)
    y2 = jnp.minimum(jnp.maximum(cy + h * 0.5, f0), _IMG)
    box_ref[0] = x1
    box_ref[1] = y1
    box_ref[2] = x2
    box_ref[3] = y2
    box_ref[4] = (x2 - x1) * (y2 - y1)

    it = jax.lax.broadcasted_iota
    rowi = (it(jnp.int32, (_T, 8, 128), 0) * 8
            + it(jnp.int32, (_T, 8, 128), 1))
    lane = it(jnp.int32, (1, 128), 1)
    lanef = lane.astype(jnp.float32)
    zleaf = jnp.full((8, 128), -jnp.inf, jnp.float32)

    def body(carry):
        n, s = carry
        x1c = box_ref[0]
        y1c = box_ref[1]
        x2c = box_ref[2]
        y2c = box_ref[3]
        items = [(s[t], rowi[t], x1c[t], y1c[t], x2c[t], y2c[t], zleaf)
                 for t in range(_T)]
        while len(items) > 1:
            nxt = [_merge(items[j], items[j + 1])
                   for j in range(0, len(items) - 1, 2)]
            if len(items) % 2:
                nxt.append(items[-1])
            items = nxt
        cur = items[0]
        for sh in (1, 2, 4):
            cur = _merge(cur, tuple(pltpu.roll(v, sh, 0) for v in cur))
        vrow = cur[0][0:1]
        v2row = cur[6][0:1]
        tx1 = cur[2][0:1]
        ty1 = cur[3][0:1]
        tx2 = cur[4][0:1]
        ty2 = cur[5][0:1]
        m = jnp.max(vrow, keepdims=True)
        mask = vrow >= m
        cnt = jnp.sum(jnp.where(mask, 1.0, f0), keepdims=True)
        x1s = jnp.sum(jnp.where(mask, tx1, f0), keepdims=True)
        y1s = jnp.sum(jnp.where(mask, ty1, f0), keepdims=True)
        x2s = jnp.sum(jnp.where(mask, tx2, f0), keepdims=True)
        y2s = jnp.sum(jnp.where(mask, ty2, f0), keepdims=True)
        notmask = jnp.where(mask, ninf, vrow)
        m2 = jnp.max(notmask, keepdims=True)
        hid1 = jnp.sum(jnp.where(mask, v2row, f0), keepdims=True)
        sel_ref[0:1, :] = jnp.broadcast_to(x1s, (1, 128))
        sel_ref[1:2, :] = jnp.broadcast_to(y1s, (1, 128))
        sel_ref[2:3, :] = jnp.broadcast_to(x2s, (1, 128))
        sel_ref[3:4, :] = jnp.broadcast_to(y2s, (1, 128))
        sel_ref[4:5, :] = jnp.broadcast_to(m, (1, 128))

        @pl.when(cnt[0, 0] > 1.0)
        def _tie_fix():
            lw = jnp.min(jnp.where(mask, lanef, 1e9), keepdims=True)
            onehot = lanef == lw
            sel_ref[0:1, :] = jnp.broadcast_to(jnp.sum(
                jnp.where(onehot, tx1, f0), keepdims=True), (1, 128))
            sel_ref[1:2, :] = jnp.broadcast_to(jnp.sum(
                jnp.where(onehot, ty1, f0), keepdims=True), (1, 128))
            sel_ref[2:3, :] = jnp.broadcast_to(jnp.sum(
                jnp.where(onehot, tx2, f0), keepdims=True), (1, 128))
            sel_ref[3:4, :] = jnp.broadcast_to(jnp.sum(
                jnp.where(onehot, ty2, f0), keepdims=True), (1, 128))
            ss_slow = jnp.where(
                m == neg,
                jnp.sum(jnp.where(lane == 0, scores_ref[0, 0:1, :], f0),
                        keepdims=True),
                m)
            sel_ref[4:5, :] = jnp.broadcast_to(ss_slow, (1, 128))

        x1b = sel_ref[0:1, :]
        y1b = sel_ref[1:2, :]
        x2b = sel_ref[2:3, :]
        y2b = sel_ref[3:4, :]
        ssb = sel_ref[4:5, :]
        area_s = (x2b - x1b) * (y2b - y1b)
        art = (tx2 - tx1) * (ty2 - ty1)
        xxr = jnp.maximum(tx1, x1b)
        yyr = jnp.maximum(ty1, y1b)
        xx2r = jnp.minimum(tx2, x2b)
        yy2r = jnp.minimum(ty2, y2b)
        interr = jnp.maximum(xx2r - xxr, f0) * jnp.maximum(yy2r - yyr, f0)
        iour = interr / (art + area_s - interr + 1e-9)
        suppr = iour >= _IOU_THR
        self_dead = jnp.sum(jnp.where(mask & suppr, 1.0, f0), keepdims=True)
        hcap = jnp.max(jnp.where(suppr & (~mask), vrow, ninf), keepdims=True)
        mask2 = (~mask) & (vrow >= m2)
        cnt2 = jnp.sum(jnp.where(mask2, 1.0, f0), keepdims=True)
        x1q = jnp.sum(jnp.where(mask2, tx1, f0), keepdims=True)
        y1q = jnp.sum(jnp.where(mask2, ty1, f0), keepdims=True)
        x2q = jnp.sum(jnp.where(mask2, tx2, f0), keepdims=True)
        y2q = jnp.sum(jnp.where(mask2, ty2, f0), keepdims=True)
        m3 = jnp.max(jnp.where(mask2, ninf, notmask), keepdims=True)
        hid2 = jnp.sum(jnp.where(mask2, v2row, f0), keepdims=True)
        ar2 = (x2q - x1q) * (y2q - y1q)
        xx12 = jnp.maximum(x1q, x1b[0:1, 0:1])
        yy12 = jnp.maximum(y1q, y1b[0:1, 0:1])
        xx22 = jnp.minimum(x2q, x2b[0:1, 0:1])
        yy22 = jnp.minimum(y2q, y2b[0:1, 0:1])
        inter12 = (jnp.maximum(xx22 - xx12, f0)
                   * jnp.maximum(yy22 - yy12, f0))
        area_s11 = area_s[0:1, 0:1]
        iou12 = inter12 / (ar2 + area_s11 - inter12 + 1e-9)
        valid2 = ((cnt == 1.0) & (cnt2 == 1.0) & (m2 > neg)
                  & (m2 > hcap) & (m2 > hid1) & (iou12 < _IOU_THR)
                  & (self_dead == 1.0))
        xxr2 = jnp.maximum(tx1, x1q)
        yyr2 = jnp.maximum(ty1, y1q)
        xx2r2 = jnp.minimum(tx2, x2q)
        yy2r2 = jnp.minimum(ty2, y2q)
        interr2 = (jnp.maximum(xx2r2 - xxr2, f0)
                   * jnp.maximum(yy2r2 - yyr2, f0))
        iour2 = interr2 / (art + ar2 - interr2 + 1e-9)
        suppr2 = iour2 >= _IOU_THR
        self_dead2 = jnp.sum(jnp.where(mask2 & suppr2, 1.0, f0),
                             keepdims=True)
        hcap3 = jnp.max(jnp.where((suppr | suppr2) & (~mask) & (~mask2),
                                  vrow, ninf), keepdims=True)
        mask3 = (~mask) & (~mask2) & (vrow >= m3)
        cnt3 = jnp.sum(jnp.where(mask3, 1.0, f0), keepdims=True)
        x1u = jnp.sum(jnp.where(mask3, tx1, f0), keepdims=True)
        y1u = jnp.sum(jnp.where(mask3, ty1, f0), keepdims=True)
        x2u = jnp.sum(jnp.where(mask3, tx2, f0), keepdims=True)
        y2u = jnp.sum(jnp.where(mask3, ty2, f0), keepdims=True)
        ar3 = (x2u - x1u) * (y2u - y1u)
        xx13 = jnp.maximum(x1u, x1b[0:1, 0:1])
        yy13 = jnp.maximum(y1u, y1b[0:1, 0:1])
        xx23 = jnp.minimum(x2u, x2b[0:1, 0:1])
        yy23 = jnp.minimum(y2u, y2b[0:1, 0:1])
        inter13 = (jnp.maximum(xx23 - xx13, f0)
                   * jnp.maximum(yy23 - yy13, f0))
        iou13 = inter13 / (ar3 + area_s11 - inter13 + 1e-9)
        xx13b = jnp.maximum(x1u, x1q)
        yy13b = jnp.maximum(y1u, y1q)
        xx23b = jnp.minimum(x2u, x2q)
        yy23b = jnp.minimum(y2u, y2q)
        inter23 = (jnp.maximum(xx23b - xx13b, f0)
                   * jnp.maximum(yy23b - yy13b, f0))
        iou23 = inter23 / (ar3 + ar2 - inter23 + 1e-9)
        valid3 = (valid2 & (cnt3 == 1.0) & (m3 > neg) & (m3 > hcap3)
                  & (m3 > hid1) & (m3 > hid2) & (iou13 < _IOU_THR)
                  & (iou23 < _IOU_THR) & (self_dead2 == 1.0))
        xx1 = jnp.maximum(x1c, x1b[None])
        yy1 = jnp.maximum(y1c, y1b[None])
        xx2 = jnp.minimum(x2c, x2b[None])
        yy2 = jnp.minimum(y2c, y2b[None])
        inter = jnp.maximum(xx2 - xx1, f0) * jnp.maximum(yy2 - yy1, f0)
        iou = inter / (box_ref[4] + area_s[None] - inter + 1e-9)
        xx1q = jnp.maximum(x1c, x1q[None])
        yy1q = jnp.maximum(y1c, y1q[None])
        xx2q = jnp.minimum(x2c, x2q[None])
        yy2q = jnp.minimum(y2c, y2q[None])
        interq = jnp.maximum(xx2q - xx1q, f0) * jnp.maximum(yy2q - yy1q, f0)
        iouq = interq / (box_ref[4] + ar2[None] - interq + 1e-9)
        xx1u = jnp.maximum(x1c, x1u[None])
        yy1u = jnp.maximum(y1c, y1u[None])
        xx2u = jnp.minimum(x2c, x2u[None])
        yy2u = jnp.minimum(y2c, y2u[None])
        interu = jnp.maximum(xx2u - xx1u, f0) * jnp.maximum(yy2u - yy1u, f0)
        iouu = interu / (box_ref[4] + ar3[None] - interu + 1e-9)
        kill = ((iou >= _IOU_THR) | (valid2[None] & (iouq >= _IOU_THR))
                | (valid3[None] & (iouu >= _IOU_THR)))
        s2 = jnp.where(kill, neg, s)
        row1 = jnp.where(lane == 0, x1b,
               jnp.where(lane == 1, y1b,
               jnp.where(lane == 2, x2b,
               jnp.where(lane == 3, y2b, ssb))))
        row2 = jnp.where(lane == 0, x1q,
               jnp.where(lane == 1, y1q,
               jnp.where(lane == 2, x2q,
               jnp.where(lane == 3, y2q, m2))))
        row3 = jnp.where(lane == 0, x1u,
               jnp.where(lane == 1, y1u,
               jnp.where(lane == 2, x2u,
               jnp.where(lane == 3, y2u, m3))))
        out_ref[pl.ds(n, 1), :] = row1
        out_ref[pl.ds(n + 1, 1), :] = row2
        out_ref[pl.ds(n + 2, 1), :] = row3
        n2 = (n + 1 + jnp.where(valid2, 1, 0)[0, 0]
              + jnp.where(valid3, 1, 0)[0, 0])
        return (n2, s2)

    jax.lax.while_loop(lambda c: c[0] < _MAX_OUT, body,
                       (jnp.int32(0), scores_ref[...]))


def _to_colmajor(a):
    return a.reshape(128, _T, 8).transpose(1, 2, 0)


def kernel(cls_output, reg_output, anchors):
    f32 = jnp.float32
    pad = _NPAD - _N
    scores = _to_colmajor(jnp.concatenate(
        [cls_output.astype(f32), jnp.full((pad,), -jnp.inf, f32)]))
    reg_p = jnp.concatenate(
        [reg_output.astype(f32), jnp.zeros((pad, 4), f32)]).T
    anc_p = jnp.concatenate(
        [anchors.astype(f32), jnp.zeros((pad, 4), f32)]).T
    reg4 = jnp.stack([_to_colmajor(reg_p[k]) for k in range(4)])
    anc4 = jnp.stack([_to_colmajor(anc_p[k]) for k in range(4)])

    out = pl.pallas_call(
        _nms_body,
        out_shape=jax.ShapeDtypeStruct((304, 128), f32),
        scratch_shapes=[
            pltpu.VMEM((5, _T, 8, 128), f32),
            pltpu.VMEM((8, 128), f32),
        ],
    )(scores, reg4, anc4)

    rois = out[:_MAX_OUT, 0:4]
    roi_scores = out[:_MAX_OUT, 4]
    return roi_scores, rois
